# Initial kernel scaffold; baseline (speedup 1.0000x reference)
import jax
import jax.numpy as jnp
from jax.experimental import pallas as pl


def kernel(x_nchw, w_hwio, bias, gamma, beta):
    raise NotImplementedError("write your pallas kernel here")


# speedup vs baseline: 1.6013x; 1.6013x over previous
import functools

import jax
import jax.numpy as jnp
from jax.experimental import pallas as pl
from jax.experimental.pallas import tpu as pltpu

EPS = 1e-5
LANES = 128


def _round_up(x, m):
    return (x + m - 1) // m * m


def _conv_stats_kernel(x_ref, w_ref, y_ref, stats_ref, *, w_out, kh_size,
                       kw_size):
    hp = x_ref.shape[1]
    h = hp - (kh_size - 1)
    cinp = x_ref.shape[3]
    rows = h * w_out
    slab = x_ref[0]
    acc = None
    for kh in range(kh_size):
        row_slab = slab[kh:kh + h]
        for kw in range(kw_size):
            win = row_slab[:, kw:kw + w_out, :]
            lhs = win.reshape(rows, cinp)
            part = jax.lax.dot_general(
                lhs, w_ref[kh * kw_size + kw],
                dimension_numbers=(((1,), (0,)), ((), ())),
                preferred_element_type=jnp.float32)
            acc = part if acc is None else acc + part
    ones_r = jnp.ones((8, rows), jnp.float32)
    dn = (((1,), (0,)), ((), ()))
    psum = jax.lax.dot_general(ones_r, acc, dn,
                               preferred_element_type=jnp.float32)
    psq = jax.lax.dot_general(ones_r, acc * acc, dn,
                              preferred_element_type=jnp.float32)
    stats_ref[0] = jnp.concatenate([psum[0:1], psq[0:1]], axis=0)
    y_ref[0] = acc[:, :y_ref.shape[2]].astype(jnp.bfloat16)


def _bn_relu_t_kernel(y_ref, scale_ref, shift_ref, o_ref):
    z = jnp.maximum(
        y_ref[0].astype(jnp.float32) * scale_ref[...] + shift_ref[...], 0.0)
    o_ref[0] = z.T


def kernel(x_nchw, w_hwio, bias, gamma, beta):
    del bias
    N, Cin, H, W = x_nchw.shape
    KH, KW, _, Cout = w_hwio.shape
    CPAD = _round_up(Cout, LANES)
    CINP = _round_up(Cin, 8)
    HP, WP = H + KH - 1, W + KW - 1
    HWROWS = H * W

    x_nhwc = jnp.transpose(x_nchw, (0, 2, 3, 1))
    x_pad = jnp.pad(x_nhwc, ((0, 0), (1, 1), (1, 1), (0, CINP - Cin))
                    ).astype(jnp.bfloat16)
    w_packed = jnp.pad(
        w_hwio.reshape(KH * KW, Cin, Cout),
        ((0, 0), (0, CINP - Cin), (0, CPAD - Cout))).astype(jnp.bfloat16)

    cparams = pltpu.CompilerParams(
        dimension_semantics=("parallel",),
        vmem_limit_bytes=64 * 1024 * 1024)

    conv_flops = 2 * N * H * W * KH * KW * CINP * CPAD
    y, stats = pl.pallas_call(
        functools.partial(_conv_stats_kernel, w_out=W, kh_size=KH,
                          kw_size=KW),
        grid=(N,),
        in_specs=[
            pl.BlockSpec((1, HP, WP, CINP), lambda n: (n, 0, 0, 0)),
            pl.BlockSpec((KH * KW, CINP, CPAD), lambda n: (0, 0, 0)),
        ],
        out_specs=[
            pl.BlockSpec((1, HWROWS, Cout), lambda n: (n, 0, 0)),
            pl.BlockSpec((1, 2, CPAD), lambda n: (n, 0, 0)),
        ],
        out_shape=[
            jax.ShapeDtypeStruct((N, HWROWS, Cout), jnp.bfloat16),
            jax.ShapeDtypeStruct((N, 2, CPAD), jnp.float32),
        ],
        compiler_params=cparams,
        cost_estimate=pl.CostEstimate(
            flops=int(conv_flops + 4 * N * HWROWS * CPAD),
            transcendentals=0,
            bytes_accessed=int(2 * (x_pad.size + w_packed.size
                                    + N * HWROWS * Cout)
                               + 4 * N * 2 * CPAD)),
    )(x_pad, w_packed)

    count = float(N * H * W)
    total = jnp.sum(stats, axis=0)
    mean = total[0, :Cout] / count
    var = total[1, :Cout] / count - mean * mean
    inv_std = jax.lax.rsqrt(var + EPS)
    scale = (gamma.astype(jnp.float32) * inv_std).reshape(1, Cout)
    shift = (beta.astype(jnp.float32) - mean * scale[0]).reshape(1, Cout)

    out = pl.pallas_call(
        _bn_relu_t_kernel,
        grid=(N,),
        in_specs=[
            pl.BlockSpec((1, HWROWS, Cout), lambda n: (n, 0, 0)),
            pl.BlockSpec((1, Cout), lambda n: (0, 0)),
            pl.BlockSpec((1, Cout), lambda n: (0, 0)),
        ],
        out_specs=pl.BlockSpec((1, Cout, HWROWS), lambda n: (n, 0, 0)),
        out_shape=jax.ShapeDtypeStruct((N, Cout, HWROWS), jnp.float32),
        compiler_params=cparams,
        cost_estimate=pl.CostEstimate(
            flops=int(3 * N * HWROWS * Cout),
            transcendentals=0,
            bytes_accessed=int(2 * N * HWROWS * Cout
                               + 4 * N * HWROWS * Cout + 8 * Cout)),
    )(y, scale, shift)

    return out.reshape(N, Cout, H, W)
